# Initial kernel scaffold; baseline (speedup 1.0000x reference)
import jax
import jax.numpy as jnp
from jax.experimental import pallas as pl


def kernel(x, coefficients_vect, scaling_coeffs_vect, grid):
    raise NotImplementedError("write your pallas kernel here")


# speedup vs baseline: 4024.0388x; 4024.0388x over previous
import functools

import jax
import jax.numpy as jnp
from jax import lax
from jax.experimental import pallas as pl
from jax.experimental.pallas import tpu as pltpu
from jax.experimental.pallas import tpu_sc as plsc

NUM_ACT = 96
SIZE = 256
B, H, W = 8, 224, 224
SLICE = H * W
NSLICES = B * NUM_ACT
NWORKERS = 32
SLICES_PER_W = NSLICES // NWORKERS
NCHUNK = 4
CHUNK = SLICE // NCHUNK
NVEC = CHUNK // 16
N = NSLICES * SLICE

_mesh = plsc.VectorSubcoreMesh(
    core_axis_name="c", subcore_axis_name="s", num_cores=2, num_subcores=16
)


@functools.partial(
    pl.kernel,
    out_type=jax.ShapeDtypeStruct((N,), jnp.float32),
    mesh=_mesh,
    compiler_params=pltpu.CompilerParams(needs_layout_passes=False),
    scratch_types=[
        pltpu.VMEM((SIZE,), jnp.float32),
        pltpu.VMEM((128,), jnp.float32),
        pltpu.VMEM((64,), jnp.float32),
        pltpu.VMEM((CHUNK,), jnp.float32),
        pltpu.VMEM((CHUNK,), jnp.float32),
    ],
)
def _spline_sc(x_hbm, coef_hbm, scal_hbm, par_hbm, out_hbm,
               tab, scal_v, par_v, xbuf, obuf):
    cid = lax.axis_index("c")
    sid = lax.axis_index("s")
    w = sid * 2 + cid

    pltpu.sync_copy(scal_hbm, scal_v.at[pl.ds(0, NUM_ACT)])
    pltpu.sync_copy(par_hbm, par_v)
    gmin = par_v[pl.ds(0, 16)]
    gmax = par_v[pl.ds(16, 16)]
    invh = par_v[pl.ds(32, 16)]
    halfr = par_v[pl.ds(48, 16)]

    def slice_body(j, _):
        sl = w * SLICES_PER_W + j
        ch = lax.rem(sl, NUM_ACT)
        pltpu.sync_copy(coef_hbm.at[pl.ds(pl.multiple_of(ch * SIZE, SIZE), SIZE)],
                        tab)
        chv = jnp.full((16,), ch, jnp.int32)
        sv = plsc.load_gather(scal_v, [chv])
        kv = halfr * sv

        def tscale(i, _):
            off = pl.multiple_of(i * 16, 16)
            tab[pl.ds(off, 16)] = tab[pl.ds(off, 16)] * sv + kv
            return 0

        lax.fori_loop(0, SIZE // 16, tscale, 0)

        def chunk_body(cix, _):
            base = pl.multiple_of(sl * SLICE + cix * CHUNK, 64)
            pltpu.sync_copy(x_hbm.at[pl.ds(base, CHUNK)], xbuf)

            def vec_body(i, _):
                off = pl.multiple_of(i * 16, 16)
                xv = xbuf[pl.ds(off, 16)]
                xc = jnp.minimum(jnp.maximum(xv, gmin), gmax)
                t = (xc - gmin) * invh
                li = jnp.minimum(t.astype(jnp.int32), SIZE - 2)
                fr = t - li.astype(jnp.float32)
                cl = plsc.load_gather(tab, [li])
                cr = plsc.load_gather(tab, [li + 1])
                obuf[pl.ds(off, 16)] = cl + fr * (cr - cl)
                return 0

            lax.fori_loop(0, NVEC, vec_body, 0)
            pltpu.sync_copy(obuf, out_hbm.at[pl.ds(base, CHUNK)])
            return 0

        lax.fori_loop(0, NCHUNK, chunk_body, 0)
        return 0

    lax.fori_loop(0, SLICES_PER_W, slice_body, 0)


def kernel(x, coefficients_vect, scaling_coeffs_vect, grid):
    xf = x.reshape(-1)
    scal = scaling_coeffs_vect.reshape(-1).astype(jnp.float32)
    gmin = grid[0]
    gmax = grid[-1]
    invh = (SIZE - 1) / (gmax - gmin)
    halfr = jnp.where(SIZE % 2 == 0, (gmax - gmin) / 2.0, 0.0)
    par = jnp.concatenate([
        jnp.full((16,), gmin, jnp.float32),
        jnp.full((16,), gmax, jnp.float32),
        jnp.full((16,), invh, jnp.float32),
        jnp.full((16,), halfr, jnp.float32),
    ])
    out = _spline_sc(xf, coefficients_vect.astype(jnp.float32), scal, par)
    return out.reshape(x.shape)


# speedup vs baseline: 7410.3834x; 1.8415x over previous
import functools

import jax
import jax.numpy as jnp
from jax import lax
from jax.experimental import pallas as pl
from jax.experimental.pallas import tpu as pltpu
from jax.experimental.pallas import tpu_sc as plsc

NUM_ACT = 96
SIZE = 256
B, H, W = 8, 224, 224
SLICE = H * W
NSLICES = B * NUM_ACT
NWORKERS = 32
SLICES_PER_W = NSLICES // NWORKERS
NCHUNK = 2
CHUNK = SLICE // NCHUNK
NVEC = CHUNK // 16
STEPS = SLICES_PER_W * NCHUNK
N = NSLICES * SLICE

_mesh = plsc.VectorSubcoreMesh(
    core_axis_name="c", subcore_axis_name="s", num_cores=2, num_subcores=16
)


@functools.partial(
    pl.kernel,
    out_type=jax.ShapeDtypeStruct((N,), jnp.float32),
    mesh=_mesh,
    compiler_params=pltpu.CompilerParams(needs_layout_passes=False),
    scratch_types=[
        pltpu.VMEM((SIZE,), jnp.float32),
        pltpu.VMEM((128,), jnp.float32),
        pltpu.VMEM((64,), jnp.float32),
        pltpu.VMEM((CHUNK,), jnp.float32),
        pltpu.VMEM((CHUNK,), jnp.float32),
        pltpu.VMEM((CHUNK,), jnp.float32),
        pltpu.VMEM((CHUNK,), jnp.float32),
        pltpu.SemaphoreType.DMA,
        pltpu.SemaphoreType.DMA,
        pltpu.SemaphoreType.DMA,
        pltpu.SemaphoreType.DMA,
    ],
)
def _spline_sc(x_hbm, coef_hbm, scal_hbm, par_hbm, out_hbm,
               tab, scal_v, par_v, xb0, xb1, ob0, ob1,
               si0, si1, so0, so1):
    cid = lax.axis_index("c")
    sid = lax.axis_index("s")
    w = sid * 2 + cid
    wbase = w * (SLICES_PER_W * SLICE)

    xb = (xb0, xb1)
    ob = (ob0, ob1)
    si = (si0, si1)
    so = (so0, so1)

    pltpu.sync_copy(scal_hbm, scal_v.at[pl.ds(0, NUM_ACT)])
    pltpu.sync_copy(par_hbm, par_v)
    gmin = par_v[pl.ds(0, 16)]
    gmax = par_v[pl.ds(16, 16)]
    invh = par_v[pl.ds(32, 16)]
    halfr = par_v[pl.ds(48, 16)]

    def in_base(g):
        return pl.multiple_of(wbase + g * CHUNK, 64)

    pltpu.async_copy(x_hbm.at[pl.ds(in_base(0), CHUNK)], xb0, si0)

    def outer(k, _):
        for b in range(2):
            g = k * 2 + b
            @pl.when(g < STEPS - 1)
            def _prefetch():
                pltpu.async_copy(
                    x_hbm.at[pl.ds(in_base(g + 1), CHUNK)], xb[1 - b], si[1 - b]
                )

            if b == 0:
                ch = lax.rem(w * SLICES_PER_W + k, NUM_ACT)
                pltpu.sync_copy(
                    coef_hbm.at[pl.ds(pl.multiple_of(ch * SIZE, SIZE), SIZE)],
                    tab,
                )
                chv = jnp.full((16,), ch, jnp.int32)
                sv = plsc.load_gather(scal_v, [chv])
                kv = halfr * sv

                @plsc.parallel_loop(0, SIZE // 16)
                def _tscale(i):
                    off = pl.multiple_of(i * 16, 16)
                    tab[pl.ds(off, 16)] = tab[pl.ds(off, 16)] * sv + kv

            pltpu.make_async_copy(
                x_hbm.at[pl.ds(0, CHUNK)], xb[b], si[b]
            ).wait()

            @pl.when(k >= 1)
            def _drain_prev():
                pltpu.make_async_copy(
                    x_hbm.at[pl.ds(0, CHUNK)], ob[b], so[b]
                ).wait()

            xbuf = xb[b]
            obuf = ob[b]

            @plsc.parallel_loop(0, NVEC, unroll=4)
            def _vec(i):
                off = pl.multiple_of(i * 16, 16)
                xv = xbuf[pl.ds(off, 16)]
                xc = jnp.minimum(jnp.maximum(xv, gmin), gmax)
                t = (xc - gmin) * invh
                li = jnp.minimum(t.astype(jnp.int32), SIZE - 2)
                fr = t - li.astype(jnp.float32)
                cl = plsc.load_gather(tab, [li])
                cr = plsc.load_gather(tab, [li + 1])
                obuf[pl.ds(off, 16)] = cl + fr * (cr - cl)

            pltpu.async_copy(obuf, out_hbm.at[pl.ds(in_base(g), CHUNK)], so[b])
        return 0

    lax.fori_loop(0, STEPS // 2, outer, 0)

    for b in range(2):
        pltpu.make_async_copy(x_hbm.at[pl.ds(0, CHUNK)], ob[b], so[b]).wait()


def kernel(x, coefficients_vect, scaling_coeffs_vect, grid):
    xf = x.reshape(-1)
    scal = scaling_coeffs_vect.reshape(-1).astype(jnp.float32)
    gmin = grid[0]
    gmax = grid[-1]
    invh = (SIZE - 1) / (gmax - gmin)
    halfr = jnp.where(SIZE % 2 == 0, (gmax - gmin) / 2.0, 0.0)
    par = jnp.concatenate([
        jnp.full((16,), gmin, jnp.float32),
        jnp.full((16,), gmax, jnp.float32),
        jnp.full((16,), invh, jnp.float32),
        jnp.full((16,), halfr, jnp.float32),
    ])
    out = _spline_sc(xf, coefficients_vect.astype(jnp.float32), scal, par)
    return out.reshape(x.shape)


# speedup vs baseline: 17431.3756x; 2.3523x over previous
import functools

import jax
import jax.numpy as jnp
from jax import lax
from jax.experimental import pallas as pl
from jax.experimental.pallas import tpu as pltpu
from jax.experimental.pallas import tpu_sc as plsc

NUM_ACT = 96
SIZE = 256
B, H, W = 8, 224, 224
NSLICES = B * NUM_ACT
NWORKERS = 32
SLICES_PER_W = NSLICES // NWORKERS
RBLK = H // 2
NVROW = W // 16

_mesh = plsc.VectorSubcoreMesh(
    core_axis_name="c", subcore_axis_name="s", num_cores=2, num_subcores=16
)


@functools.partial(
    pl.kernel,
    out_type=jax.ShapeDtypeStruct((NSLICES, H, W), jnp.float32),
    mesh=_mesh,
    compiler_params=pltpu.CompilerParams(needs_layout_passes=False),
    scratch_types=[
        pltpu.VMEM((SIZE,), jnp.float32),
        pltpu.VMEM((128,), jnp.float32),
        pltpu.VMEM((64,), jnp.float32),
        pltpu.VMEM((RBLK, W), jnp.float32),
        pltpu.VMEM((RBLK, W), jnp.float32),
        pltpu.VMEM((RBLK, W), jnp.float32),
        pltpu.VMEM((RBLK, W), jnp.float32),
        pltpu.SemaphoreType.DMA,
        pltpu.SemaphoreType.DMA,
        pltpu.SemaphoreType.DMA,
        pltpu.SemaphoreType.DMA,
    ],
)
def _spline_sc(x_hbm, coef_hbm, scal_hbm, par_hbm, out_hbm,
               tab, scal_v, par_v, xb0, xb1, ob0, ob1,
               si0, si1, so0, so1):
    cid = lax.axis_index("c")
    sid = lax.axis_index("s")
    w = sid * 2 + cid
    sl0 = w * SLICES_PER_W

    xb = (xb0, xb1)
    ob = (ob0, ob1)
    si = (si0, si1)
    so = (so0, so1)

    pltpu.sync_copy(scal_hbm, scal_v.at[pl.ds(0, NUM_ACT)])
    pltpu.sync_copy(par_hbm, par_v)
    gmin = par_v[pl.ds(0, 16)]
    gmax = par_v[pl.ds(16, 16)]
    invh = par_v[pl.ds(32, 16)]
    halfr = par_v[pl.ds(48, 16)]

    pltpu.async_copy(x_hbm.at[sl0, pl.ds(0, RBLK)], xb0, si0)

    def outer(k, _):
        sl = sl0 + k
        for b in range(2):
            if b == 0:
                pltpu.async_copy(x_hbm.at[sl, pl.ds(RBLK, RBLK)], xb1, si1)
            else:
                @pl.when(k < SLICES_PER_W - 1)
                def _prefetch():
                    pltpu.async_copy(x_hbm.at[sl + 1, pl.ds(0, RBLK)], xb0, si0)

            if b == 0:
                ch = lax.rem(sl, NUM_ACT)
                pltpu.sync_copy(
                    coef_hbm.at[pl.ds(pl.multiple_of(ch * SIZE, SIZE), SIZE)],
                    tab,
                )
                chv = jnp.full((16,), ch, jnp.int32)
                sv = plsc.load_gather(scal_v, [chv])
                kv = halfr * sv

                @plsc.parallel_loop(0, SIZE // 16)
                def _tscale(i):
                    off = pl.multiple_of(i * 16, 16)
                    tab[pl.ds(off, 16)] = tab[pl.ds(off, 16)] * sv + kv

            pltpu.make_async_copy(
                x_hbm.at[0, pl.ds(0, RBLK)], xb[b], si[b]
            ).wait()

            @pl.when(k >= 1)
            def _drain_prev():
                pltpu.make_async_copy(
                    x_hbm.at[0, pl.ds(0, RBLK)], ob[b], so[b]
                ).wait()

            xbuf = xb[b]
            obuf = ob[b]

            @plsc.parallel_loop(0, RBLK)
            def _row(r):
                for i in range(NVROW):
                    off = i * 16
                    xv = xbuf[r, pl.ds(off, 16)]
                    xc = jnp.minimum(jnp.maximum(xv, gmin), gmax)
                    t = (xc - gmin) * invh
                    li = jnp.minimum(t.astype(jnp.int32), SIZE - 2)
                    fr = t - li.astype(jnp.float32)
                    cl = plsc.load_gather(tab, [li])
                    cr = plsc.load_gather(tab, [li + 1])
                    obuf[r, pl.ds(off, 16)] = cl + fr * (cr - cl)

            pltpu.async_copy(obuf, out_hbm.at[sl, pl.ds(b * RBLK, RBLK)], so[b])
        return 0

    lax.fori_loop(0, SLICES_PER_W, outer, 0)

    for b in range(2):
        pltpu.make_async_copy(x_hbm.at[0, pl.ds(0, RBLK)], ob[b], so[b]).wait()


def kernel(x, coefficients_vect, scaling_coeffs_vect, grid):
    x3 = x.reshape(NSLICES, H, W)
    scal = scaling_coeffs_vect.reshape(-1).astype(jnp.float32)
    gmin = grid[0]
    gmax = grid[-1]
    invh = (SIZE - 1) / (gmax - gmin)
    halfr = jnp.where(SIZE % 2 == 0, (gmax - gmin) / 2.0, 0.0)
    par = jnp.concatenate([
        jnp.full((16,), gmin, jnp.float32),
        jnp.full((16,), gmax, jnp.float32),
        jnp.full((16,), invh, jnp.float32),
        jnp.full((16,), halfr, jnp.float32),
    ])
    out = _spline_sc(x3, coefficients_vect.astype(jnp.float32), scal, par)
    return out.reshape(x.shape)
